# baseline (speedup 1.0000x reference)
import functools

import jax
import jax.numpy as jnp
from jax import lax
from jax.experimental import pallas as pl
from jax.experimental.pallas import tpu as pltpu
from jax.experimental.pallas import tpu_sc as plsc

F32 = jnp.float32


def _rup(x, m):
    return (x + m - 1) // m * m


def _pick_bm(m):
    if m % 1000 == 0:
        return 1000
    return m


def _mm(x, w, b=None, act=None):
    m0, k = x.shape
    n = w.shape[1]
    m = _rup(m0, 8)
    if m != m0:
        x = jnp.pad(x, ((0, m - m0), (0, 0)))
    bm = _pick_bm(m)
    grid = (m // bm,)
    in_specs = [pl.BlockSpec((bm, k), lambda i: (i, 0)),
                pl.BlockSpec((k, n), lambda i: (0, 0))]
    args = [x, w]
    has_b = b is not None
    if has_b:
        in_specs.append(pl.BlockSpec((1, n), lambda i: (0, 0)))
        args.append(b.reshape(1, n))

    def body(*refs):
        x_ref, w_ref = refs[0], refs[1]
        o_ref = refs[-1]
        acc = jnp.dot(x_ref[...], w_ref[...], preferred_element_type=F32)
        if has_b:
            acc = acc + refs[2][...]
        if act is not None:
            acc = act(acc)
        o_ref[...] = acc

    out = pl.pallas_call(
        body, grid=grid, in_specs=in_specs,
        out_specs=pl.BlockSpec((bm, n), lambda i: (i, 0)),
        out_shape=jax.ShapeDtypeStruct((m, n), F32))(*args)
    return out[:m0] if m != m0 else out


def _update(den, num, xd, w1a, w1b, b1, w2, b2, ndst, extra=None):
    c_dim, m, s = den.shape
    t = xd.shape[1]
    o = w2.shape[0]
    if xd.shape[0] != m:
        xd = jnp.pad(xd, ((0, m - xd.shape[0]), (0, 0)))
    if extra is not None and extra.shape[0] != m:
        extra = jnp.pad(extra, ((0, m - extra.shape[0]), (0, 0)))
    bm = _pick_bm(m)
    grid = (m // bm,)
    specs = [pl.BlockSpec((c_dim, bm, s), lambda i: (0, i, 0)),
             pl.BlockSpec((c_dim, bm, s), lambda i: (0, i, 0)),
             pl.BlockSpec((bm, t), lambda i: (i, 0)),
             pl.BlockSpec((s, o), lambda i: (0, 0)),
             pl.BlockSpec((t, o), lambda i: (0, 0)),
             pl.BlockSpec((1, o), lambda i: (0, 0)),
             pl.BlockSpec((o, o), lambda i: (0, 0)),
             pl.BlockSpec((1, o), lambda i: (0, 0))]
    args = [den, num, xd, w1a, w1b, b1.reshape(1, o), w2, b2.reshape(1, o)]
    has_extra = extra is not None
    if has_extra:
        specs.append(pl.BlockSpec((bm, o), lambda i: (i, 0)))
        args.append(extra)

    def body(*refs):
        den_ref, num_ref, xd_ref = refs[0], refs[1], refs[2]
        w1a_ref, w1b_ref, b1_ref, w2_ref, b2_ref = refs[3:8]
        o_ref = refs[-1]
        dt = den_ref[0]
        nt = num_ref[0]
        for c in range(1, c_dim):
            dt = dt + den_ref[c]
            nt = nt + num_ref[c]
        aggr = nt / jnp.maximum(dt, 1e-16)
        h = jnp.tanh(jnp.dot(aggr, w1a_ref[...], preferred_element_type=F32)
                     + jnp.dot(xd_ref[...], w1b_ref[...], preferred_element_type=F32)
                     + b1_ref[...])
        res = jnp.tanh(jnp.dot(h, w2_ref[...], preferred_element_type=F32)
                       + b2_ref[...])
        if has_extra:
            res = res + refs[8][...]
        o_ref[...] = res

    out = pl.pallas_call(
        body, grid=grid, in_specs=specs,
        out_specs=pl.BlockSpec((bm, o), lambda i: (i, 0)),
        out_shape=jax.ShapeDtypeStruct((m, o), F32))(*args)
    return out[:ndst]


_NC, _NS, _L = 2, 16, 16


def _sc_cfg(nsrc, ndst, s, e_cnt):
    budget = 3900 * 1024
    fc = 64
    nh = 1
    while True:
        ndp = _rup(ndst + 1, 128 * nh)
        hd = ndp // nh
        accr = (hd + 128) if nh > 1 else ndp
        if accr * 2 * fc * 4 <= budget:
            break
        nh *= 2
    ncha = s // fc
    bw = 128
    epw0 = _rup(-(-e_cnt // 32), 8)
    cb = min(128, epw0)
    if epw0 <= cb:
        epw, nk = cb, 1
    else:
        epw = _rup(epw0, 2 * cb)
        nk = epw // cb
    return fc, ncha, ndp, bw, cb, nk, epw, nh, hd, accr


@functools.lru_cache(maxsize=None)
def _make_edge_kernel(nsrc, ndst, s, e_cnt):
    fc, ncha, ndp, bw, cb, nk, epw, nh, hd, accr = _sc_cfg(nsrc, ndst, s, e_cnt)
    npk = _rup(s, 128) // bw
    nrz = accr // _NS
    nrh = hd // _NS
    mesh = plsc.VectorSubcoreMesh(core_axis_name="c", subcore_axis_name="s")

    def body(*refs):
        sx_chunks = refs[0:ncha]
        b_packs = refs[ncha:ncha + npk]
        srci, dsti, zeros = refs[ncha + npk:ncha + npk + 3]
        out = refs[ncha + npk + 3]
        (si, db, di2, sxv, bv, res,
         accum, sem) = refs[ncha + npk + 4:]
        if nh == 1:
            di2 = db
        cid = lax.axis_index("c")
        sid = lax.axis_index("s")
        wid = cid * _NS + sid

        def process(h, off):

            @plsc.parallel_loop(0, cb, 1, unroll=2)
            def edge(e_idx):
                for f in range(fc // _L):
                    asl = pl.ds(f * _L, _L)
                    xsl = pl.ds(fc + f * _L, _L)
                    z = sxv[e_idx, asl] + bv[e_idx, pl.ds(off + f * _L, _L)]
                    att = 1.0 / (1.0 + jnp.exp(-z))
                    msg = att * sxv[e_idx, xsl]
                    ex = jnp.exp(msg)
                    res[e_idx, asl] = ex
                    res[e_idx, xsl] = ex * msg

            pltpu.sync_copy(res, accum.at[di2], add=True)

        for f_i in range(ncha):
            pidx = f_i * fc // bw
            off = f_i * fc % bw
            for h in range(nh):
                pltpu.sync_copy(zeros.at[pl.ds(sid * nrz, nrz)],
                                accum.at[pl.ds(sid * nrz, nrz)])
                plsc.subcore_barrier()
                def ebody(k, carry):
                    base = wid * epw + k * cb
                    pltpu.sync_copy(srci.at[pl.ds(base, cb)], si)
                    pltpu.sync_copy(dsti.at[pl.ds(base, cb)], db)
                    c1 = pltpu.async_copy(sx_chunks[f_i].at[si], sxv, sem)
                    c2 = pltpu.async_copy(b_packs[pidx].at[db], bv, sem)
                    c1.wait()
                    c2.wait()
                    if nh > 1:
                        for j in range(cb // _L):
                            jsl = pl.ds(j * _L, _L)
                            v = db[jsl]
                            li = v - (h * hd)
                            okm = (li >= 0) & (li < hd)
                            spread = hd + ((j * _L +
                                            lax.iota(jnp.int32, 16)) & 127)
                            di2[jsl] = jnp.where(okm, li, spread)
                    process(h, off)
                    return carry

                lax.fori_loop(0, nk, ebody, 0)
                plsc.subcore_barrier()
                row0 = (cid * ncha + f_i) * ndp + h * hd + sid * nrh
                pltpu.sync_copy(accum.at[pl.ds(sid * nrh, nrh)],
                                out.at[pl.ds(row0, nrh)])
                plsc.subcore_barrier()

    scratch = [
        pltpu.VMEM((cb,), jnp.int32),
        pltpu.VMEM((cb,), jnp.int32),
        pltpu.VMEM((cb,), jnp.int32),
        pltpu.VMEM((cb, 2 * fc), F32),
        pltpu.VMEM((cb, bw), F32),
        pltpu.VMEM((cb, 2 * fc), F32),
        pltpu.VMEM_SHARED((accr, 2 * fc), F32),
        pltpu.SemaphoreType.DMA,
    ]
    return functools.partial(
        pl.kernel, mesh=mesh,
        out_type=jax.ShapeDtypeStruct((_NC * ncha * ndp, 2 * fc), F32),
        scratch_types=scratch)(body)


def _edge_sc(asrc, adst, xsrc, src, dst, tok):
    nsrc, s = asrc.shape
    ndst = adst.shape[0]
    e_cnt = src.shape[0]
    fc, ncha, ndp, bw, cb, nk, epw, nh, hd, accr = _sc_cfg(nsrc, ndst, s,
                                                           e_cnt)
    npk = _rup(s, 128) // bw
    epad = 32 * epw
    asp = jnp.pad(asrc, ((0, 1), (0, 0)))
    adp = jnp.pad(adst, ((0, 1), (0, _rup(s, 128) - s)))
    xsp = jnp.pad(xsrc, ((0, 1), (0, 0)))
    srcp = jnp.concatenate(
        [src, jnp.full((epad - e_cnt,), nsrc, jnp.int32)])
    dstp = jnp.concatenate(
        [dst, jnp.full((epad - e_cnt,), ndst, jnp.int32)])
    zeros = jnp.zeros((accr, 2 * fc), F32) + tok * 0.0
    sx_chunks = [
        jnp.concatenate([asp[:, i * fc:(i + 1) * fc],
                         xsp[:, i * fc:(i + 1) * fc]], axis=1)
        for i in range(ncha)]
    b_packs = [adp[:, p * bw:(p + 1) * bw] for p in range(npk)]
    kern = _make_edge_kernel(nsrc, ndst, s, e_cnt)
    out = kern(*sx_chunks, *b_packs, srcp, dstp, zeros)
    out4 = out.reshape(_NC, ncha, ndp, 2 * fc)
    nd8 = _rup(ndst, 8)
    den = out4[:, :, :nd8, :fc].transpose(0, 2, 1, 3).reshape(_NC, nd8, s)
    num = out4[:, :, :nd8, fc:].transpose(0, 2, 1, 3).reshape(_NC, nd8, s)
    return den, num, out[0, 0]


def _attn(prm, xs, xd, ei, tok, extra=None):
    we, be, w1, b1, w2, b2 = prm
    s = xs.shape[1]
    t = xd.shape[1]
    ndst = xd.shape[0]
    adst = _mm(xd, we[:t], be)
    asrc = _mm(xs, we[t:])
    den, num, tok = _edge_sc(asrc, adst, xs, ei[0], ei[1], tok)
    return _update(den, num, xd, w1[:s], w1[s:], b1, w2, b2, ndst, extra), tok


def kernel(x_u, x_v, x_y, x_sp, x_oph, x_pmt, x_opf, x_evt, ei_plane_u, ei_nexus_u, ei_sp_plane_u, ei_plane_v, ei_nexus_v, ei_sp_plane_v, ei_plane_y, ei_nexus_y, ei_sp_plane_y, ei_sumpe, ei_flash, ei_sp_evt, ei_opf_evt, ei_evt_sp, ei_evt_opf, ei_opf_pmt, ei_pmt_oph, plane_net_u_We, plane_net_u_be, plane_net_u_W1, plane_net_u_b1, plane_net_u_W2, plane_net_u_b2, p2n_u_We, p2n_u_be, p2n_u_W1, p2n_u_b1, p2n_u_W2, p2n_u_b2, n2p_u_We, n2p_u_be, n2p_u_W1, n2p_u_b1, n2p_u_W2, n2p_u_b2, plane_net_v_We, plane_net_v_be, plane_net_v_W1, plane_net_v_b1, plane_net_v_W2, plane_net_v_b2, p2n_v_We, p2n_v_be, p2n_v_W1, p2n_v_b1, p2n_v_W2, p2n_v_b2, n2p_v_We, n2p_v_be, n2p_v_W1, n2p_v_b1, n2p_v_W2, n2p_v_b2, plane_net_y_We, plane_net_y_be, plane_net_y_W1, plane_net_y_b1, plane_net_y_W2, plane_net_y_b2, p2n_y_We, p2n_y_be, p2n_y_W1, p2n_y_b1, p2n_y_W2, p2n_y_b2, n2p_y_We, n2p_y_be, n2p_y_W1, n2p_y_b1, n2p_y_W2, n2p_y_b2, h2pmt_We, h2pmt_be, h2pmt_W1, h2pmt_b1, h2pmt_W2, h2pmt_b2, pmt2f_We, pmt2f_be, pmt2f_W1, pmt2f_b1, pmt2f_W2, pmt2f_b2, n2i_We, n2i_be, n2i_W1, n2i_b1, n2i_W2, n2i_b2, f2i_We, f2i_be, f2i_W1, f2i_b1, f2i_W2, f2i_b2, i2n_We, i2n_be, i2n_W1, i2n_b1, i2n_W2, i2n_b2, i2f_We, i2f_be, i2f_W1, i2f_b1, i2f_W2, i2f_b2, f2pmt_We, f2pmt_be, f2pmt_W1, f2pmt_b1, f2pmt_W2, f2pmt_b2, pmt2oph_We, pmt2oph_be, pmt2oph_W1, pmt2oph_b1, pmt2oph_W2, pmt2oph_b2):
    planes = {
        "u": (x_u, ei_plane_u, ei_nexus_u, ei_sp_plane_u,
              (plane_net_u_We, plane_net_u_be, plane_net_u_W1,
               plane_net_u_b1, plane_net_u_W2, plane_net_u_b2),
              (p2n_u_We, p2n_u_be, p2n_u_W1, p2n_u_b1, p2n_u_W2, p2n_u_b2),
              (n2p_u_We, n2p_u_be, n2p_u_W1, n2p_u_b1, n2p_u_W2, n2p_u_b2)),
        "v": (x_v, ei_plane_v, ei_nexus_v, ei_sp_plane_v,
              (plane_net_v_We, plane_net_v_be, plane_net_v_W1,
               plane_net_v_b1, plane_net_v_W2, plane_net_v_b2),
              (p2n_v_We, p2n_v_be, p2n_v_W1, p2n_v_b1, p2n_v_W2, p2n_v_b2),
              (n2p_v_We, n2p_v_be, n2p_v_W1, n2p_v_b1, n2p_v_W2, n2p_v_b2)),
        "y": (x_y, ei_plane_y, ei_nexus_y, ei_sp_plane_y,
              (plane_net_y_We, plane_net_y_be, plane_net_y_W1,
               plane_net_y_b1, plane_net_y_W2, plane_net_y_b2),
              (p2n_y_We, p2n_y_be, p2n_y_W1, p2n_y_b1, p2n_y_W2, p2n_y_b2),
              (n2p_y_We, n2p_y_be, n2p_y_W1, n2p_y_b1, n2p_y_W2, n2p_y_b2)),
    }
    tok = jnp.float32(0.0)
    p = {}
    for k, (x_pl, ei_pl, _, _, prm_pl, _, _) in planes.items():
        p[k], tok = _attn(prm_pl, x_pl, x_pl, ei_pl, tok)
    n_parts = []
    for k in ("u", "v", "y"):
        part, tok = _attn(planes[k][5], p[k], x_sp, planes[k][2], tok)
        n_parts.append(part)
    n = jnp.concatenate(n_parts, axis=1)
    pmt, tok = _attn((h2pmt_We, h2pmt_be, h2pmt_W1, h2pmt_b1, h2pmt_W2,
                      h2pmt_b2), x_oph, x_pmt, ei_sumpe, tok)
    opf, tok = _attn((pmt2f_We, pmt2f_be, pmt2f_W1, pmt2f_b1, pmt2f_W2,
                      pmt2f_b2), pmt, x_opf, ei_flash, tok)
    evt_a, tok = _attn((n2i_We, n2i_be, n2i_W1, n2i_b1, n2i_W2, n2i_b2),
                       n, x_evt, ei_sp_evt, tok)
    evt, tok = _attn((f2i_We, f2i_be, f2i_W1, f2i_b1, f2i_W2, f2i_b2),
                     opf, x_evt, ei_opf_evt, tok, extra=evt_a)
    n2, tok = _attn((i2n_We, i2n_be, i2n_W1, i2n_b1, i2n_W2, i2n_b2),
                    evt, n, ei_evt_sp, tok)
    p2 = {}
    for k in ("u", "v", "y"):
        p2[k], tok = _attn(planes[k][6], n2, p[k], planes[k][3], tok)
    opf2, tok = _attn((i2f_We, i2f_be, i2f_W1, i2f_b1, i2f_W2, i2f_b2),
                      evt, opf, ei_evt_opf, tok)
    pmt2, tok = _attn((f2pmt_We, f2pmt_be, f2pmt_W1, f2pmt_b1, f2pmt_W2,
                       f2pmt_b2), opf2, pmt, ei_opf_pmt, tok)
    oph2, tok = _attn((pmt2oph_We, pmt2oph_be, pmt2oph_W1, pmt2oph_b1,
                       pmt2oph_W2, pmt2oph_b2), pmt2, x_oph, ei_pmt_oph, tok)
    return (p2["u"], p2["v"], p2["y"], n2, oph2, pmt2, opf2, evt)


# speedup vs baseline: 1.2735x; 1.2735x over previous
import functools

import jax
import jax.numpy as jnp
from jax import lax
from jax.experimental import pallas as pl
from jax.experimental.pallas import tpu as pltpu
from jax.experimental.pallas import tpu_sc as plsc

F32 = jnp.float32


def _rup(x, m):
    return (x + m - 1) // m * m


def _pick_bm(m):
    if m % 1000 == 0:
        return 1000
    return m


def _mm(x, w, b=None, act=None):
    m0, k = x.shape
    n = w.shape[1]
    m = _rup(m0, 8)
    if m != m0:
        x = jnp.pad(x, ((0, m - m0), (0, 0)))
    bm = _pick_bm(m)
    grid = (m // bm,)
    in_specs = [pl.BlockSpec((bm, k), lambda i: (i, 0)),
                pl.BlockSpec((k, n), lambda i: (0, 0))]
    args = [x, w]
    has_b = b is not None
    if has_b:
        in_specs.append(pl.BlockSpec((1, n), lambda i: (0, 0)))
        args.append(b.reshape(1, n))

    def body(*refs):
        x_ref, w_ref = refs[0], refs[1]
        o_ref = refs[-1]
        acc = jnp.dot(x_ref[...], w_ref[...], preferred_element_type=F32)
        if has_b:
            acc = acc + refs[2][...]
        if act is not None:
            acc = act(acc)
        o_ref[...] = acc

    out = pl.pallas_call(
        body, grid=grid, in_specs=in_specs,
        out_specs=pl.BlockSpec((bm, n), lambda i: (i, 0)),
        out_shape=jax.ShapeDtypeStruct((m, n), F32))(*args)
    return out[:m0] if m != m0 else out


def _update(den, num, xd, w1a, w1b, b1, w2, b2, ndst, extra=None):
    c_dim, m, s = den.shape
    t = xd.shape[1]
    o = w2.shape[0]
    if xd.shape[0] != m:
        xd = jnp.pad(xd, ((0, m - xd.shape[0]), (0, 0)))
    if extra is not None and extra.shape[0] != m:
        extra = jnp.pad(extra, ((0, m - extra.shape[0]), (0, 0)))
    bm = _pick_bm(m)
    grid = (m // bm,)
    specs = [pl.BlockSpec((c_dim, bm, s), lambda i: (0, i, 0)),
             pl.BlockSpec((c_dim, bm, s), lambda i: (0, i, 0)),
             pl.BlockSpec((bm, t), lambda i: (i, 0)),
             pl.BlockSpec((s, o), lambda i: (0, 0)),
             pl.BlockSpec((t, o), lambda i: (0, 0)),
             pl.BlockSpec((1, o), lambda i: (0, 0)),
             pl.BlockSpec((o, o), lambda i: (0, 0)),
             pl.BlockSpec((1, o), lambda i: (0, 0))]
    args = [den, num, xd, w1a, w1b, b1.reshape(1, o), w2, b2.reshape(1, o)]
    has_extra = extra is not None
    if has_extra:
        specs.append(pl.BlockSpec((bm, o), lambda i: (i, 0)))
        args.append(extra)

    def body(*refs):
        den_ref, num_ref, xd_ref = refs[0], refs[1], refs[2]
        w1a_ref, w1b_ref, b1_ref, w2_ref, b2_ref = refs[3:8]
        o_ref = refs[-1]
        dt = den_ref[0]
        nt = num_ref[0]
        for c in range(1, c_dim):
            dt = dt + den_ref[c]
            nt = nt + num_ref[c]
        aggr = nt / jnp.maximum(dt, 1e-16)
        h = jnp.tanh(jnp.dot(aggr, w1a_ref[...], preferred_element_type=F32)
                     + jnp.dot(xd_ref[...], w1b_ref[...], preferred_element_type=F32)
                     + b1_ref[...])
        res = jnp.tanh(jnp.dot(h, w2_ref[...], preferred_element_type=F32)
                       + b2_ref[...])
        if has_extra:
            res = res + refs[8][...]
        o_ref[...] = res

    out = pl.pallas_call(
        body, grid=grid, in_specs=specs,
        out_specs=pl.BlockSpec((bm, o), lambda i: (i, 0)),
        out_shape=jax.ShapeDtypeStruct((m, o), F32))(*args)
    return out[:ndst]


_NC, _NS, _L = 2, 16, 16


def _sc_cfg(nsrc, ndst, s, e_cnt):
    budget = 3900 * 1024
    fc = 64
    nh = 1
    while True:
        ndp = _rup(ndst + 1, 128 * nh)
        hd = ndp // nh
        accr = (hd + 128) if nh > 1 else ndp
        if accr * 2 * fc * 4 <= budget:
            break
        nh *= 2
    ncha = s // fc
    bw = 128
    epw0 = _rup(-(-e_cnt // 32), 8)
    cb = min(128, epw0)
    epw = _rup(epw0, cb)
    nk = epw // cb
    return fc, ncha, ndp, bw, cb, nk, epw, nh, hd, accr


@functools.lru_cache(maxsize=None)
def _make_edge_kernel(nsrc, ndst, s, e_cnt):
    fc, ncha, ndp, bw, cb, nk, epw, nh, hd, accr = _sc_cfg(nsrc, ndst, s, e_cnt)
    npk = _rup(s, 128) // bw
    nrz = accr // _NS
    nrh = hd // _NS
    mesh = plsc.VectorSubcoreMesh(core_axis_name="c", subcore_axis_name="s")

    def body(*refs):
        sx_chunks = refs[0:ncha]
        b_packs = refs[ncha:ncha + npk]
        srci, dsti, zeros = refs[ncha + npk:ncha + npk + 3]
        out = refs[ncha + npk + 3]
        (si, db, di2, sxv, bv, res,
         accum, sem) = refs[ncha + npk + 4:]
        if nh == 1:
            di2 = db
        cid = lax.axis_index("c")
        sid = lax.axis_index("s")
        wid = cid * _NS + sid

        def process(h, off):

            def edge(e_idx, cr):
                for f in range(fc // _L):
                    asl = pl.ds(f * _L, _L)
                    xsl = pl.ds(fc + f * _L, _L)
                    z = sxv[e_idx, asl] + bv[e_idx, pl.ds(off + f * _L, _L)]
                    att = 1.0 / (1.0 + jnp.exp(-z))
                    msg = att * sxv[e_idx, xsl]
                    ex = jnp.exp(msg)
                    res[e_idx, asl] = ex
                    res[e_idx, xsl] = ex * msg
                return cr

            lax.fori_loop(0, cb, edge, 0)
            pltpu.sync_copy(res, accum.at[di2], add=True)

        for f_i in range(ncha):
            pidx = f_i * fc // bw
            off = f_i * fc % bw
            for h in range(nh):
                pltpu.sync_copy(zeros.at[pl.ds(sid * nrz, nrz)],
                                accum.at[pl.ds(sid * nrz, nrz)])
                plsc.subcore_barrier()
                def ebody(k, carry):
                    base = wid * epw + k * cb
                    pltpu.sync_copy(srci.at[pl.ds(base, cb)], si)
                    pltpu.sync_copy(dsti.at[pl.ds(base, cb)], db)
                    c1 = pltpu.async_copy(sx_chunks[f_i].at[si], sxv, sem)
                    c2 = pltpu.async_copy(b_packs[pidx].at[db], bv, sem)
                    c1.wait()
                    c2.wait()
                    if nh > 1:
                        for j in range(cb // _L):
                            jsl = pl.ds(j * _L, _L)
                            v = db[jsl]
                            li = v - (h * hd)
                            okm = (li >= 0) & (li < hd)
                            spread = hd + ((j * _L +
                                            lax.iota(jnp.int32, 16)) & 127)
                            di2[jsl] = jnp.where(okm, li, spread)
                    process(h, off)
                    return carry

                lax.fori_loop(0, nk, ebody, 0)
                plsc.subcore_barrier()
                row0 = (cid * ncha + f_i) * ndp + h * hd + sid * nrh
                pltpu.sync_copy(accum.at[pl.ds(sid * nrh, nrh)],
                                out.at[pl.ds(row0, nrh)])
                plsc.subcore_barrier()

    scratch = [
        pltpu.VMEM((cb,), jnp.int32),
        pltpu.VMEM((cb,), jnp.int32),
        pltpu.VMEM((cb,), jnp.int32),
        pltpu.VMEM((cb, 2 * fc), F32),
        pltpu.VMEM((cb, bw), F32),
        pltpu.VMEM((cb, 2 * fc), F32),
        pltpu.VMEM_SHARED((accr, 2 * fc), F32),
        pltpu.SemaphoreType.DMA,
    ]
    return functools.partial(
        pl.kernel, mesh=mesh,
        out_type=jax.ShapeDtypeStruct((_NC * ncha * ndp, 2 * fc), F32),
        scratch_types=scratch)(body)


def _edge_sc(asrc, adst, xsrc, src, dst, tok):
    nsrc, s = asrc.shape
    ndst = adst.shape[0]
    e_cnt = src.shape[0]
    fc, ncha, ndp, bw, cb, nk, epw, nh, hd, accr = _sc_cfg(nsrc, ndst, s,
                                                           e_cnt)
    npk = _rup(s, 128) // bw
    epad = 32 * epw
    asp = jnp.pad(asrc, ((0, 1), (0, 0)))
    adp = jnp.pad(adst, ((0, 1), (0, _rup(s, 128) - s)))
    xsp = jnp.pad(xsrc, ((0, 1), (0, 0)))
    srcp = jnp.concatenate(
        [src, jnp.full((epad - e_cnt,), nsrc, jnp.int32)])
    dstp = jnp.concatenate(
        [dst, jnp.full((epad - e_cnt,), ndst, jnp.int32)])
    zeros = jnp.zeros((accr, 2 * fc), F32) + tok * 0.0
    sx_chunks = [
        jnp.concatenate([asp[:, i * fc:(i + 1) * fc],
                         xsp[:, i * fc:(i + 1) * fc]], axis=1)
        for i in range(ncha)]
    b_packs = [adp[:, p * bw:(p + 1) * bw] for p in range(npk)]
    kern = _make_edge_kernel(nsrc, ndst, s, e_cnt)
    out = kern(*sx_chunks, *b_packs, srcp, dstp, zeros)
    out4 = out.reshape(_NC, ncha, ndp, 2 * fc)
    nd8 = _rup(ndst, 8)
    den = out4[:, :, :nd8, :fc].transpose(0, 2, 1, 3).reshape(_NC, nd8, s)
    num = out4[:, :, :nd8, fc:].transpose(0, 2, 1, 3).reshape(_NC, nd8, s)
    return den, num, out[0, 0]


def _attn(prm, xs, xd, ei, tok, extra=None):
    we, be, w1, b1, w2, b2 = prm
    s = xs.shape[1]
    t = xd.shape[1]
    ndst = xd.shape[0]
    adst = _mm(xd, we[:t], be)
    asrc = _mm(xs, we[t:])
    den, num, tok = _edge_sc(asrc, adst, xs, ei[0], ei[1], tok)
    return _update(den, num, xd, w1[:s], w1[s:], b1, w2, b2, ndst, extra), tok


def kernel(x_u, x_v, x_y, x_sp, x_oph, x_pmt, x_opf, x_evt, ei_plane_u, ei_nexus_u, ei_sp_plane_u, ei_plane_v, ei_nexus_v, ei_sp_plane_v, ei_plane_y, ei_nexus_y, ei_sp_plane_y, ei_sumpe, ei_flash, ei_sp_evt, ei_opf_evt, ei_evt_sp, ei_evt_opf, ei_opf_pmt, ei_pmt_oph, plane_net_u_We, plane_net_u_be, plane_net_u_W1, plane_net_u_b1, plane_net_u_W2, plane_net_u_b2, p2n_u_We, p2n_u_be, p2n_u_W1, p2n_u_b1, p2n_u_W2, p2n_u_b2, n2p_u_We, n2p_u_be, n2p_u_W1, n2p_u_b1, n2p_u_W2, n2p_u_b2, plane_net_v_We, plane_net_v_be, plane_net_v_W1, plane_net_v_b1, plane_net_v_W2, plane_net_v_b2, p2n_v_We, p2n_v_be, p2n_v_W1, p2n_v_b1, p2n_v_W2, p2n_v_b2, n2p_v_We, n2p_v_be, n2p_v_W1, n2p_v_b1, n2p_v_W2, n2p_v_b2, plane_net_y_We, plane_net_y_be, plane_net_y_W1, plane_net_y_b1, plane_net_y_W2, plane_net_y_b2, p2n_y_We, p2n_y_be, p2n_y_W1, p2n_y_b1, p2n_y_W2, p2n_y_b2, n2p_y_We, n2p_y_be, n2p_y_W1, n2p_y_b1, n2p_y_W2, n2p_y_b2, h2pmt_We, h2pmt_be, h2pmt_W1, h2pmt_b1, h2pmt_W2, h2pmt_b2, pmt2f_We, pmt2f_be, pmt2f_W1, pmt2f_b1, pmt2f_W2, pmt2f_b2, n2i_We, n2i_be, n2i_W1, n2i_b1, n2i_W2, n2i_b2, f2i_We, f2i_be, f2i_W1, f2i_b1, f2i_W2, f2i_b2, i2n_We, i2n_be, i2n_W1, i2n_b1, i2n_W2, i2n_b2, i2f_We, i2f_be, i2f_W1, i2f_b1, i2f_W2, i2f_b2, f2pmt_We, f2pmt_be, f2pmt_W1, f2pmt_b1, f2pmt_W2, f2pmt_b2, pmt2oph_We, pmt2oph_be, pmt2oph_W1, pmt2oph_b1, pmt2oph_W2, pmt2oph_b2):
    planes = {
        "u": (x_u, ei_plane_u, ei_nexus_u, ei_sp_plane_u,
              (plane_net_u_We, plane_net_u_be, plane_net_u_W1,
               plane_net_u_b1, plane_net_u_W2, plane_net_u_b2),
              (p2n_u_We, p2n_u_be, p2n_u_W1, p2n_u_b1, p2n_u_W2, p2n_u_b2),
              (n2p_u_We, n2p_u_be, n2p_u_W1, n2p_u_b1, n2p_u_W2, n2p_u_b2)),
        "v": (x_v, ei_plane_v, ei_nexus_v, ei_sp_plane_v,
              (plane_net_v_We, plane_net_v_be, plane_net_v_W1,
               plane_net_v_b1, plane_net_v_W2, plane_net_v_b2),
              (p2n_v_We, p2n_v_be, p2n_v_W1, p2n_v_b1, p2n_v_W2, p2n_v_b2),
              (n2p_v_We, n2p_v_be, n2p_v_W1, n2p_v_b1, n2p_v_W2, n2p_v_b2)),
        "y": (x_y, ei_plane_y, ei_nexus_y, ei_sp_plane_y,
              (plane_net_y_We, plane_net_y_be, plane_net_y_W1,
               plane_net_y_b1, plane_net_y_W2, plane_net_y_b2),
              (p2n_y_We, p2n_y_be, p2n_y_W1, p2n_y_b1, p2n_y_W2, p2n_y_b2),
              (n2p_y_We, n2p_y_be, n2p_y_W1, n2p_y_b1, n2p_y_W2, n2p_y_b2)),
    }
    tok = jnp.float32(0.0)
    p = {}
    for k, (x_pl, ei_pl, _, _, prm_pl, _, _) in planes.items():
        p[k], tok = _attn(prm_pl, x_pl, x_pl, ei_pl, tok)
    n_parts = []
    for k in ("u", "v", "y"):
        part, tok = _attn(planes[k][5], p[k], x_sp, planes[k][2], tok)
        n_parts.append(part)
    n = jnp.concatenate(n_parts, axis=1)
    pmt, tok = _attn((h2pmt_We, h2pmt_be, h2pmt_W1, h2pmt_b1, h2pmt_W2,
                      h2pmt_b2), x_oph, x_pmt, ei_sumpe, tok)
    opf, tok = _attn((pmt2f_We, pmt2f_be, pmt2f_W1, pmt2f_b1, pmt2f_W2,
                      pmt2f_b2), pmt, x_opf, ei_flash, tok)
    evt_a, tok = _attn((n2i_We, n2i_be, n2i_W1, n2i_b1, n2i_W2, n2i_b2),
                       n, x_evt, ei_sp_evt, tok)
    evt, tok = _attn((f2i_We, f2i_be, f2i_W1, f2i_b1, f2i_W2, f2i_b2),
                     opf, x_evt, ei_opf_evt, tok, extra=evt_a)
    n2, tok = _attn((i2n_We, i2n_be, i2n_W1, i2n_b1, i2n_W2, i2n_b2),
                    evt, n, ei_evt_sp, tok)
    p2 = {}
    for k in ("u", "v", "y"):
        p2[k], tok = _attn(planes[k][6], n2, p[k], planes[k][3], tok)
    opf2, tok = _attn((i2f_We, i2f_be, i2f_W1, i2f_b1, i2f_W2, i2f_b2),
                      evt, opf, ei_evt_opf, tok)
    pmt2, tok = _attn((f2pmt_We, f2pmt_be, f2pmt_W1, f2pmt_b1, f2pmt_W2,
                       f2pmt_b2), opf2, pmt, ei_opf_pmt, tok)
    oph2, tok = _attn((pmt2oph_We, pmt2oph_be, pmt2oph_W1, pmt2oph_b1,
                       pmt2oph_W2, pmt2oph_b2), pmt2, x_oph, ei_pmt_oph, tok)
    return (p2["u"], p2["v"], p2["y"], n2, oph2, pmt2, opf2, evt)
